# baseline (speedup 1.0000x reference)
import functools

import jax
import jax.numpy as jnp
from jax import lax
from jax.experimental import pallas as pl
from jax.experimental.pallas import tpu as pltpu
from jax.experimental.pallas import tpu_sc as plsc

_ROWS = 16384
_LANES = 128
_BR = 2048
_NB_TC = 5
_R0 = _NB_TC * _BR

_NW = 32
_PR = 32
_PIECE = _PR * _LANES
_SC_ROWS = _ROWS - _R0
_WROWS = _SC_ROWS // 16
_NPIECES = _WROWS // _PR


def _tc_kernel(pred_ref, dose_ref, mask_ref, ptv_ref, oar_ref,
               out_ref, s_ref, mx_ref, mn_ref):
    b = pl.program_id(0)
    r = pl.program_id(1)

    @pl.when(jnp.logical_and(b == 0, r == 0))
    def _init():
        s_ref[...] = jnp.zeros_like(s_ref)
        mx_ref[...] = jnp.full_like(mx_ref, -jnp.inf)
        mn_ref[...] = jnp.full_like(mn_ref, jnp.inf)

    pred = pred_ref[0]
    dose = dose_ref[0, 0]
    mask = mask_ref[0, 0]
    ptv = ptv_ref[0]

    mb = mask > 0.0
    m = mb.astype(jnp.float32)
    mp = (ptv > 0.0).astype(jnp.float32)
    oar_sum = jnp.sum(oar_ref[0], axis=0)
    mo = (oar_sum > 0.0).astype(jnp.float32)

    d = jnp.abs(pred - dose)

    def tile_sum(x):
        return jnp.sum(x.reshape(_BR // 8, 8, _LANES), axis=0)

    s_ref[0] += tile_sum(d * m)
    s_ref[1] += tile_sum(m)
    s_ref[2] += tile_sum(d * mp)
    s_ref[3] += tile_sum(mp)
    s_ref[4] += tile_sum(d * mo)
    s_ref[5] += tile_sum(mo)

    neg_inf = jnp.float32(-jnp.inf)
    pos_inf = jnp.float32(jnp.inf)

    def tile_max(x):
        return jnp.max(x.reshape(_BR // 8, 8, _LANES), axis=0)

    def tile_min(x):
        return jnp.min(x.reshape(_BR // 8, 8, _LANES), axis=0)

    mx_ref[0] = jnp.maximum(mx_ref[0], tile_max(jnp.where(mb, dose, neg_inf)))
    mx_ref[1] = jnp.maximum(mx_ref[1], tile_max(jnp.where(mb, pred, neg_inf)))
    mn_ref[0] = jnp.minimum(mn_ref[0], tile_min(jnp.where(mb, dose, pos_inf)))
    mn_ref[1] = jnp.minimum(mn_ref[1], tile_min(jnp.where(mb, pred, pos_inf)))

    @pl.when(jnp.logical_and(b == pl.num_programs(0) - 1,
                             r == pl.num_programs(1) - 1))
    def _finalize():
        for q in range(6):
            out_ref[q] = jnp.sum(s_ref[q])
        out_ref[6] = jnp.max(mx_ref[0])
        out_ref[7] = jnp.max(mx_ref[1])
        out_ref[8] = jnp.min(mn_ref[0])
        out_ref[9] = jnp.min(mn_ref[1])


def _tc_partials(pred3, gt4, ptv3, oar4):
    return pl.pallas_call(
        _tc_kernel,
        grid=(2, _NB_TC),
        in_specs=[
            pl.BlockSpec((1, _BR, _LANES), lambda b, r: (b, r, 0)),
            pl.BlockSpec((1, 1, _BR, _LANES), lambda b, r: (0, b, r, 0)),
            pl.BlockSpec((1, 1, _BR, _LANES), lambda b, r: (1, b, r, 0)),
            pl.BlockSpec((1, _BR, _LANES), lambda b, r: (b, r, 0)),
            pl.BlockSpec((1, 7, _BR, _LANES), lambda b, r: (b, 0, r, 0)),
        ],
        out_specs=pl.BlockSpec(memory_space=pltpu.SMEM),
        out_shape=jax.ShapeDtypeStruct((10,), jnp.float32),
        scratch_shapes=[
            pltpu.VMEM((6, 8, _LANES), jnp.float32),
            pltpu.VMEM((2, 8, _LANES), jnp.float32),
            pltpu.VMEM((2, 8, _LANES), jnp.float32),
        ],
    )(pred3, gt4, gt4, ptv3, oar4)


def _sc_body(pred_hbm, gt_hbm, ptv_hbm, oar_hbm, out_hbm,
             bp, bd, bm, bt, bo0, bo1, bo2, bo3, bo4, bo5, bo6, bout):
    wid = lax.axis_index("s") * 2 + lax.axis_index("c")
    b = wid // 16
    wsub = wid % 16
    row0 = _R0 + wsub * _WROWS

    zero = jnp.zeros((16,), jnp.float32)
    init = (zero, zero, zero, zero, zero, zero,
            jnp.full((16,), -jnp.inf, jnp.float32),
            jnp.full((16,), -jnp.inf, jnp.float32),
            jnp.full((16,), jnp.inf, jnp.float32),
            jnp.full((16,), jnp.inf, jnp.float32))

    oar_bufs = (bo0, bo1, bo2, bo3, bo4, bo5, bo6)

    def piece_body(p, accs):
        off = row0 + p * _PR
        base = (b * _ROWS + off) * _LANES
        pltpu.sync_copy(pred_hbm.at[pl.ds(base, _PIECE)], bp)
        pltpu.sync_copy(gt_hbm.at[pl.ds(base, _PIECE)], bd)
        pltpu.sync_copy(gt_hbm.at[pl.ds(base + 2 * _ROWS * _LANES, _PIECE)], bm)
        pltpu.sync_copy(ptv_hbm.at[pl.ds(base, _PIECE)], bt)
        for c in range(7):
            oar_base = ((b * 7 + c) * _ROWS + off) * _LANES
            pltpu.sync_copy(oar_hbm.at[pl.ds(oar_base, _PIECE)], oar_bufs[c])

        def vec_body(j, accs):
            (s1, c1, s2, c2, s3, c3, mxd, mxp, mnd, mnp) = accs
            sl = pl.ds(j * 16, 16)
            pv = bp[sl]
            dv = bd[sl]
            mb = bm[sl] > 0.0
            tb = bt[sl] > 0.0
            ov = bo0[sl] + bo1[sl] + bo2[sl] + bo3[sl] + bo4[sl] \
                + bo5[sl] + bo6[sl]
            ob = ov > 0.0

            d = jnp.abs(pv - dv)
            one = jnp.full((16,), 1.0, jnp.float32)
            zz = jnp.zeros((16,), jnp.float32)

            s1 = s1 + jnp.where(mb, d, zz)
            c1 = c1 + jnp.where(mb, one, zz)
            s2 = s2 + jnp.where(tb, d, zz)
            c2 = c2 + jnp.where(tb, one, zz)
            s3 = s3 + jnp.where(ob, d, zz)
            c3 = c3 + jnp.where(ob, one, zz)

            ninf = jnp.full((16,), -jnp.inf, jnp.float32)
            pinf = jnp.full((16,), jnp.inf, jnp.float32)
            mxd = jnp.maximum(mxd, jnp.where(mb, dv, ninf))
            mxp = jnp.maximum(mxp, jnp.where(mb, pv, ninf))
            mnd = jnp.minimum(mnd, jnp.where(mb, dv, pinf))
            mnp = jnp.minimum(mnp, jnp.where(mb, pv, pinf))
            return (s1, c1, s2, c2, s3, c3, mxd, mxp, mnd, mnp)

        return lax.fori_loop(0, _PIECE // 16, vec_body, accs)

    accs = lax.fori_loop(0, _NPIECES, piece_body, init)

    for q in range(10):
        bout[q] = accs[q]
    pltpu.sync_copy(bout, out_hbm.at[wid])


def _sc_partials(pred_flat, gt_flat, ptv_flat, oar_flat):
    mesh = plsc.VectorSubcoreMesh(core_axis_name="c", subcore_axis_name="s")
    scratch = [pltpu.VMEM((_PIECE,), jnp.float32) for _ in range(11)]
    scratch.append(pltpu.VMEM((10, 16), jnp.float32))
    fn = pl.kernel(
        _sc_body,
        mesh=mesh,
        out_type=jax.ShapeDtypeStruct((_NW, 10, 16), jnp.float32),
        scratch_types=scratch,
    )
    return fn(pred_flat, gt_flat, ptv_flat, oar_flat)


@jax.jit
def kernel(pred, gt, PTVs, OAR, max_dose_weight, min_dose_weight, PTV_weight):
    pred3 = pred.reshape(2, _ROWS, _LANES)
    gt4 = gt.reshape(2, 2, _ROWS, _LANES)
    ptv3 = PTVs.reshape(2, _ROWS, _LANES)
    oar4 = OAR.reshape(2, 7, _ROWS, _LANES)

    tc = _tc_partials(pred3, gt4, ptv3, oar4)
    sc = _sc_partials(pred.reshape(-1), gt.reshape(-1),
                      PTVs.reshape(-1), OAR.reshape(-1))

    s1 = tc[0] + jnp.sum(sc[:, 0, :])
    c1 = tc[1] + jnp.sum(sc[:, 1, :])
    s2 = tc[2] + jnp.sum(sc[:, 2, :])
    c2 = tc[3] + jnp.sum(sc[:, 3, :])
    s3 = tc[4] + jnp.sum(sc[:, 4, :])
    c3 = tc[5] + jnp.sum(sc[:, 5, :])
    dose_max = jnp.maximum(tc[6], jnp.max(sc[:, 6, :]))
    pred_max = jnp.maximum(tc[7], jnp.max(sc[:, 7, :]))
    dose_min = jnp.minimum(tc[8], jnp.min(sc[:, 8, :]))
    pred_min = jnp.minimum(tc[9], jnp.min(sc[:, 9, :]))

    max_pen = jnp.maximum(pred_max - dose_max, 0.0) ** 2
    min_pen = jnp.maximum(dose_min - pred_min, 0.0) ** 2
    return (s1 / c1
            + PTV_weight * (s2 / c2)
            + s3 / c3
            + max_dose_weight * max_pen
            + min_dose_weight * min_pen)


# speedup vs baseline: 1.8181x; 1.8181x over previous
import functools

import jax
import jax.numpy as jnp
from jax import lax
from jax.experimental import pallas as pl
from jax.experimental.pallas import tpu as pltpu
from jax.experimental.pallas import tpu_sc as plsc

_ROWS = 16384
_LANES = 128
_BR = 2048
_NB_TC = 5
_R0 = _NB_TC * _BR

_NW = 32
_PR = 32
_PIECE = _PR * _LANES
_SC_ROWS = _ROWS - _R0
_WROWS = _SC_ROWS // 16
_NPIECES = _WROWS // _PR


def _tc_kernel(pred_ref, dose_ref, mask_ref, ptv_ref, oar_ref,
               out_ref, s_ref, mx_ref, mn_ref):
    b = pl.program_id(0)
    r = pl.program_id(1)

    @pl.when(jnp.logical_and(b == 0, r == 0))
    def _init():
        s_ref[...] = jnp.zeros_like(s_ref)
        mx_ref[...] = jnp.full_like(mx_ref, -jnp.inf)
        mn_ref[...] = jnp.full_like(mn_ref, jnp.inf)

    pred = pred_ref[0]
    dose = dose_ref[0, 0]
    mask = mask_ref[0, 0]
    ptv = ptv_ref[0]

    mb = mask > 0.0
    m = mb.astype(jnp.float32)
    mp = (ptv > 0.0).astype(jnp.float32)
    oar_sum = jnp.sum(oar_ref[0], axis=0)
    mo = (oar_sum > 0.0).astype(jnp.float32)

    d = jnp.abs(pred - dose)

    def tile_sum(x):
        return jnp.sum(x.reshape(_BR // 8, 8, _LANES), axis=0)

    s_ref[0] += tile_sum(d * m)
    s_ref[1] += tile_sum(m)
    s_ref[2] += tile_sum(d * mp)
    s_ref[3] += tile_sum(mp)
    s_ref[4] += tile_sum(d * mo)
    s_ref[5] += tile_sum(mo)

    neg_inf = jnp.float32(-jnp.inf)
    pos_inf = jnp.float32(jnp.inf)

    def tile_max(x):
        return jnp.max(x.reshape(_BR // 8, 8, _LANES), axis=0)

    def tile_min(x):
        return jnp.min(x.reshape(_BR // 8, 8, _LANES), axis=0)

    mx_ref[0] = jnp.maximum(mx_ref[0], tile_max(jnp.where(mb, dose, neg_inf)))
    mx_ref[1] = jnp.maximum(mx_ref[1], tile_max(jnp.where(mb, pred, neg_inf)))
    mn_ref[0] = jnp.minimum(mn_ref[0], tile_min(jnp.where(mb, dose, pos_inf)))
    mn_ref[1] = jnp.minimum(mn_ref[1], tile_min(jnp.where(mb, pred, pos_inf)))

    @pl.when(jnp.logical_and(b == pl.num_programs(0) - 1,
                             r == pl.num_programs(1) - 1))
    def _finalize():
        for q in range(6):
            out_ref[q] = jnp.sum(s_ref[q])
        out_ref[6] = jnp.max(mx_ref[0])
        out_ref[7] = jnp.max(mx_ref[1])
        out_ref[8] = jnp.min(mn_ref[0])
        out_ref[9] = jnp.min(mn_ref[1])


def _tc_partials(pred3, gt4, ptv3, oar4):
    return pl.pallas_call(
        _tc_kernel,
        grid=(2, _NB_TC),
        in_specs=[
            pl.BlockSpec((1, _BR, _LANES), lambda b, r: (b, r, 0)),
            pl.BlockSpec((1, 1, _BR, _LANES), lambda b, r: (0, b, r, 0)),
            pl.BlockSpec((1, 1, _BR, _LANES), lambda b, r: (1, b, r, 0)),
            pl.BlockSpec((1, _BR, _LANES), lambda b, r: (b, r, 0)),
            pl.BlockSpec((1, 7, _BR, _LANES), lambda b, r: (b, 0, r, 0)),
        ],
        out_specs=pl.BlockSpec(memory_space=pltpu.SMEM),
        out_shape=jax.ShapeDtypeStruct((10,), jnp.float32),
        scratch_shapes=[
            pltpu.VMEM((6, 8, _LANES), jnp.float32),
            pltpu.VMEM((2, 8, _LANES), jnp.float32),
            pltpu.VMEM((2, 8, _LANES), jnp.float32),
        ],
    )(pred3, gt4, gt4, ptv3, oar4)


def _sc_body(pred_hbm, gt_hbm, ptv_hbm, oar_hbm, out_hbm,
             bp, bd, bm, bt, bo, bout, sem0, sem1):
    wid = lax.axis_index("s") * 2 + lax.axis_index("c")
    b = wid // 16
    wsub = wid % 16
    row0 = _R0 + wsub * _WROWS

    sems = (sem0, sem1)

    def piece_copies(p, slot):
        off = row0 + p * _PR
        base = (b * _ROWS + off) * _LANES
        sem = sems[slot]
        cps = [
            pltpu.make_async_copy(pred_hbm.at[pl.ds(base, _PIECE)],
                                  bp.at[pl.ds(slot * _PIECE, _PIECE)], sem),
            pltpu.make_async_copy(gt_hbm.at[pl.ds(base, _PIECE)],
                                  bd.at[pl.ds(slot * _PIECE, _PIECE)], sem),
            pltpu.make_async_copy(
                gt_hbm.at[pl.ds(base + 2 * _ROWS * _LANES, _PIECE)],
                bm.at[pl.ds(slot * _PIECE, _PIECE)], sem),
            pltpu.make_async_copy(ptv_hbm.at[pl.ds(base, _PIECE)],
                                  bt.at[pl.ds(slot * _PIECE, _PIECE)], sem),
        ]
        for c in range(7):
            oar_base = ((b * 7 + c) * _ROWS + off) * _LANES
            cps.append(pltpu.make_async_copy(
                oar_hbm.at[pl.ds(oar_base, _PIECE)],
                bo.at[pl.ds((slot * 7 + c) * _PIECE, _PIECE)], sem))
        return cps

    def issue(p, slot):
        for cp in piece_copies(p, slot):
            cp.start()

    def drain(p, slot):
        for cp in piece_copies(p, slot):
            cp.wait()

    def compute(slot, accs):
        def vec_body(j, accs):
            (s1, c1, s2, c2, s3, c3, mxd, mxp, mnd, mnp) = accs
            o = slot * _PIECE + j * 16
            sl = pl.ds(o, 16)
            pv = bp[sl]
            dv = bd[sl]
            mb = bm[sl] > 0.0
            tb = bt[sl] > 0.0
            oo = slot * 7 * _PIECE + j * 16
            ov = bo[pl.ds(oo, 16)] + bo[pl.ds(oo + _PIECE, 16)] \
                + bo[pl.ds(oo + 2 * _PIECE, 16)] + bo[pl.ds(oo + 3 * _PIECE, 16)] \
                + bo[pl.ds(oo + 4 * _PIECE, 16)] + bo[pl.ds(oo + 5 * _PIECE, 16)] \
                + bo[pl.ds(oo + 6 * _PIECE, 16)]
            ob = ov > 0.0

            d = jnp.abs(pv - dv)
            one = jnp.full((16,), 1.0, jnp.float32)
            zz = jnp.zeros((16,), jnp.float32)

            s1 = s1 + jnp.where(mb, d, zz)
            c1 = c1 + jnp.where(mb, one, zz)
            s2 = s2 + jnp.where(tb, d, zz)
            c2 = c2 + jnp.where(tb, one, zz)
            s3 = s3 + jnp.where(ob, d, zz)
            c3 = c3 + jnp.where(ob, one, zz)

            ninf = jnp.full((16,), -jnp.inf, jnp.float32)
            pinf = jnp.full((16,), jnp.inf, jnp.float32)
            mxd = jnp.maximum(mxd, jnp.where(mb, dv, ninf))
            mxp = jnp.maximum(mxp, jnp.where(mb, pv, ninf))
            mnd = jnp.minimum(mnd, jnp.where(mb, dv, pinf))
            mnp = jnp.minimum(mnp, jnp.where(mb, pv, pinf))
            return (s1, c1, s2, c2, s3, c3, mxd, mxp, mnd, mnp)

        return lax.fori_loop(0, _PIECE // 16, vec_body, accs, unroll=2)

    zero = jnp.zeros((16,), jnp.float32)
    accs = (zero, zero, zero, zero, zero, zero,
            jnp.full((16,), -jnp.inf, jnp.float32),
            jnp.full((16,), -jnp.inf, jnp.float32),
            jnp.full((16,), jnp.inf, jnp.float32),
            jnp.full((16,), jnp.inf, jnp.float32))

    issue(0, 0)

    def pair_body(p2, accs):
        p = 2 * p2
        issue(p + 1, 1)
        drain(p, 0)
        accs = compute(0, accs)

        @pl.when(p2 < _NPIECES // 2 - 1)
        def _():
            issue(p + 2, 0)

        drain(p + 1, 1)
        return compute(1, accs)

    accs = lax.fori_loop(0, _NPIECES // 2, pair_body, accs)

    for q in range(10):
        bout[q] = accs[q]
    pltpu.sync_copy(bout, out_hbm.at[wid])


def _sc_partials(pred_flat, gt_flat, ptv_flat, oar_flat):
    mesh = plsc.VectorSubcoreMesh(core_axis_name="c", subcore_axis_name="s")
    scratch = [
        pltpu.VMEM((2 * _PIECE,), jnp.float32),
        pltpu.VMEM((2 * _PIECE,), jnp.float32),
        pltpu.VMEM((2 * _PIECE,), jnp.float32),
        pltpu.VMEM((2 * _PIECE,), jnp.float32),
        pltpu.VMEM((2 * 7 * _PIECE,), jnp.float32),
        pltpu.VMEM((10, 16), jnp.float32),
        pltpu.SemaphoreType.DMA,
        pltpu.SemaphoreType.DMA,
    ]
    fn = pl.kernel(
        _sc_body,
        mesh=mesh,
        out_type=jax.ShapeDtypeStruct((_NW, 10, 16), jnp.float32),
        scratch_types=scratch,
    )
    return fn(pred_flat, gt_flat, ptv_flat, oar_flat)


@jax.jit
def kernel(pred, gt, PTVs, OAR, max_dose_weight, min_dose_weight, PTV_weight):
    pred3 = pred.reshape(2, _ROWS, _LANES)
    gt4 = gt.reshape(2, 2, _ROWS, _LANES)
    ptv3 = PTVs.reshape(2, _ROWS, _LANES)
    oar4 = OAR.reshape(2, 7, _ROWS, _LANES)

    sc = _sc_partials(pred.reshape(-1), gt.reshape(-1),
                      PTVs.reshape(-1), OAR.reshape(-1))
    tc = _tc_partials(pred3, gt4, ptv3, oar4)

    s1 = tc[0] + jnp.sum(sc[:, 0, :])
    c1 = tc[1] + jnp.sum(sc[:, 1, :])
    s2 = tc[2] + jnp.sum(sc[:, 2, :])
    c2 = tc[3] + jnp.sum(sc[:, 3, :])
    s3 = tc[4] + jnp.sum(sc[:, 4, :])
    c3 = tc[5] + jnp.sum(sc[:, 5, :])
    dose_max = jnp.maximum(tc[6], jnp.max(sc[:, 6, :]))
    pred_max = jnp.maximum(tc[7], jnp.max(sc[:, 7, :]))
    dose_min = jnp.minimum(tc[8], jnp.min(sc[:, 8, :]))
    pred_min = jnp.minimum(tc[9], jnp.min(sc[:, 9, :]))

    max_pen = jnp.maximum(pred_max - dose_max, 0.0) ** 2
    min_pen = jnp.maximum(dose_min - pred_min, 0.0) ** 2
    return (s1 / c1
            + PTV_weight * (s2 / c2)
            + s3 / c3
            + max_dose_weight * max_pen
            + min_dose_weight * min_pen)


# speedup vs baseline: 1.8232x; 1.0028x over previous
import functools

import jax
import jax.numpy as jnp
from jax import lax
from jax.experimental import pallas as pl
from jax.experimental.pallas import tpu as pltpu
from jax.experimental.pallas import tpu_sc as plsc

_ROWS = 16384
_LANES = 128
_BR = 2048
_NB_TC = 5
_R0 = _NB_TC * _BR

_NW = 32
_PR = 32
_PIECE = _PR * _LANES
_SC_ROWS = _ROWS - _R0
_WROWS = _SC_ROWS // 16
_NPIECES = _WROWS // _PR


def _tc_kernel(pred_ref, dose_ref, mask_ref, ptv_ref, oar_ref,
               out_ref, s_ref, mx_ref, mn_ref):
    b = pl.program_id(0)
    r = pl.program_id(1)

    @pl.when(jnp.logical_and(b == 0, r == 0))
    def _init():
        s_ref[...] = jnp.zeros_like(s_ref)
        mx_ref[...] = jnp.full_like(mx_ref, -jnp.inf)
        mn_ref[...] = jnp.full_like(mn_ref, jnp.inf)

    pred = pred_ref[0]
    dose = dose_ref[0, 0]
    mask = mask_ref[0, 0]
    ptv = ptv_ref[0]

    mb = mask > 0.0
    m = mb.astype(jnp.float32)
    mp = (ptv > 0.0).astype(jnp.float32)
    oar_sum = jnp.sum(oar_ref[0], axis=0)
    mo = (oar_sum > 0.0).astype(jnp.float32)

    d = jnp.abs(pred - dose)

    def tile_sum(x):
        return jnp.sum(x.reshape(_BR // 8, 8, _LANES), axis=0)

    s_ref[0] += tile_sum(d * m)
    s_ref[1] += tile_sum(m)
    s_ref[2] += tile_sum(d * mp)
    s_ref[3] += tile_sum(mp)
    s_ref[4] += tile_sum(d * mo)
    s_ref[5] += tile_sum(mo)

    neg_inf = jnp.float32(-jnp.inf)
    pos_inf = jnp.float32(jnp.inf)

    def tile_max(x):
        return jnp.max(x.reshape(_BR // 8, 8, _LANES), axis=0)

    def tile_min(x):
        return jnp.min(x.reshape(_BR // 8, 8, _LANES), axis=0)

    mx_ref[0] = jnp.maximum(mx_ref[0], tile_max(jnp.where(mb, dose, neg_inf)))
    mx_ref[1] = jnp.maximum(mx_ref[1], tile_max(jnp.where(mb, pred, neg_inf)))
    mn_ref[0] = jnp.minimum(mn_ref[0], tile_min(jnp.where(mb, dose, pos_inf)))
    mn_ref[1] = jnp.minimum(mn_ref[1], tile_min(jnp.where(mb, pred, pos_inf)))

    @pl.when(jnp.logical_and(b == pl.num_programs(0) - 1,
                             r == pl.num_programs(1) - 1))
    def _finalize():
        for q in range(6):
            out_ref[q] = jnp.sum(s_ref[q])
        out_ref[6] = jnp.max(mx_ref[0])
        out_ref[7] = jnp.max(mx_ref[1])
        out_ref[8] = jnp.min(mn_ref[0])
        out_ref[9] = jnp.min(mn_ref[1])


def _tc_partials(pred3, gt4, ptv3, oar4):
    return pl.pallas_call(
        _tc_kernel,
        grid=(2, _NB_TC),
        in_specs=[
            pl.BlockSpec((1, _BR, _LANES), lambda b, r: (b, r, 0)),
            pl.BlockSpec((1, 1, _BR, _LANES), lambda b, r: (0, b, r, 0)),
            pl.BlockSpec((1, 1, _BR, _LANES), lambda b, r: (1, b, r, 0)),
            pl.BlockSpec((1, _BR, _LANES), lambda b, r: (b, r, 0)),
            pl.BlockSpec((1, 7, _BR, _LANES), lambda b, r: (b, 0, r, 0)),
        ],
        out_specs=pl.BlockSpec(memory_space=pltpu.SMEM),
        out_shape=jax.ShapeDtypeStruct((10,), jnp.float32),
        scratch_shapes=[
            pltpu.VMEM((6, 8, _LANES), jnp.float32),
            pltpu.VMEM((2, 8, _LANES), jnp.float32),
            pltpu.VMEM((2, 8, _LANES), jnp.float32),
        ],
    )(pred3, gt4, gt4, ptv3, oar4)


def _sc_body(pred_hbm, gt_hbm, ptv_hbm, oar_hbm, out_hbm,
             bp, bd, bm, bt, bo, bout, sem0, sem1):
    wid = lax.axis_index("s") * 2 + lax.axis_index("c")
    b = wid // 16
    wsub = wid % 16
    row0 = _R0 + wsub * _WROWS

    sems = (sem0, sem1)

    def piece_copies(p, slot):
        off = row0 + p * _PR
        base = (b * _ROWS + off) * _LANES
        sem = sems[slot]
        cps = [
            pltpu.make_async_copy(pred_hbm.at[pl.ds(base, _PIECE)],
                                  bp.at[pl.ds(slot * _PIECE, _PIECE)], sem),
            pltpu.make_async_copy(gt_hbm.at[pl.ds(base, _PIECE)],
                                  bd.at[pl.ds(slot * _PIECE, _PIECE)], sem),
            pltpu.make_async_copy(
                gt_hbm.at[pl.ds(base + 2 * _ROWS * _LANES, _PIECE)],
                bm.at[pl.ds(slot * _PIECE, _PIECE)], sem),
            pltpu.make_async_copy(ptv_hbm.at[pl.ds(base, _PIECE)],
                                  bt.at[pl.ds(slot * _PIECE, _PIECE)], sem),
        ]
        for c in range(7):
            oar_base = ((b * 7 + c) * _ROWS + off) * _LANES
            cps.append(pltpu.make_async_copy(
                oar_hbm.at[pl.ds(oar_base, _PIECE)],
                bo.at[pl.ds((slot * 7 + c) * _PIECE, _PIECE)], sem))
        return cps

    def issue(p, slot):
        for cp in piece_copies(p, slot):
            cp.start()

    def drain(p, slot):
        for cp in piece_copies(p, slot):
            cp.wait()

    def compute(slot, accs):
        def vec_body(j, accs):
            (s1, c1, s2, c2, s3, c3, mxd, mxp, mnd, mnp) = accs
            o = slot * _PIECE + j * 16
            sl = pl.ds(o, 16)
            pv = bp[sl]
            dv = bd[sl]
            mb = bm[sl] > 0.0
            tb = bt[sl] > 0.0
            oo = slot * 7 * _PIECE + j * 16
            ov = bo[pl.ds(oo, 16)] + bo[pl.ds(oo + _PIECE, 16)] \
                + bo[pl.ds(oo + 2 * _PIECE, 16)] + bo[pl.ds(oo + 3 * _PIECE, 16)] \
                + bo[pl.ds(oo + 4 * _PIECE, 16)] + bo[pl.ds(oo + 5 * _PIECE, 16)] \
                + bo[pl.ds(oo + 6 * _PIECE, 16)]
            ob = ov > 0.0

            d = jnp.abs(pv - dv)
            one = jnp.full((16,), 1.0, jnp.float32)
            zz = jnp.zeros((16,), jnp.float32)

            s1 = s1 + jnp.where(mb, d, zz)
            c1 = c1 + jnp.where(mb, one, zz)
            s2 = s2 + jnp.where(tb, d, zz)
            c2 = c2 + jnp.where(tb, one, zz)
            s3 = s3 + jnp.where(ob, d, zz)
            c3 = c3 + jnp.where(ob, one, zz)

            ninf = jnp.full((16,), -jnp.inf, jnp.float32)
            pinf = jnp.full((16,), jnp.inf, jnp.float32)
            mxd = jnp.maximum(mxd, jnp.where(mb, dv, ninf))
            mxp = jnp.maximum(mxp, jnp.where(mb, pv, ninf))
            mnd = jnp.minimum(mnd, jnp.where(mb, dv, pinf))
            mnp = jnp.minimum(mnp, jnp.where(mb, pv, pinf))
            return (s1, c1, s2, c2, s3, c3, mxd, mxp, mnd, mnp)

        return lax.fori_loop(0, _PIECE // 16, vec_body, accs, unroll=2)

    zero = jnp.zeros((16,), jnp.float32)
    accs = (zero, zero, zero, zero, zero, zero,
            jnp.full((16,), -jnp.inf, jnp.float32),
            jnp.full((16,), -jnp.inf, jnp.float32),
            jnp.full((16,), jnp.inf, jnp.float32),
            jnp.full((16,), jnp.inf, jnp.float32))

    issue(0, 0)

    def pair_body(p2, accs):
        p = 2 * p2
        issue(p + 1, 1)
        drain(p, 0)
        accs = compute(0, accs)

        @pl.when(p2 < _NPIECES // 2 - 1)
        def _():
            issue(p + 2, 0)

        drain(p + 1, 1)
        return compute(1, accs)

    accs = lax.fori_loop(0, _NPIECES // 2, pair_body, accs)

    for q in range(10):
        bout[q] = accs[q]
    pltpu.sync_copy(bout, out_hbm.at[wid])


def _sc_partials(pred_flat, gt_flat, ptv_flat, oar_flat):
    mesh = plsc.VectorSubcoreMesh(core_axis_name="c", subcore_axis_name="s")
    scratch = [
        pltpu.VMEM((2 * _PIECE,), jnp.float32),
        pltpu.VMEM((2 * _PIECE,), jnp.float32),
        pltpu.VMEM((2 * _PIECE,), jnp.float32),
        pltpu.VMEM((2 * _PIECE,), jnp.float32),
        pltpu.VMEM((2 * 7 * _PIECE,), jnp.float32),
        pltpu.VMEM((10, 16), jnp.float32),
        pltpu.SemaphoreType.DMA,
        pltpu.SemaphoreType.DMA,
    ]
    fn = pl.kernel(
        _sc_body,
        mesh=mesh,
        out_type=jax.ShapeDtypeStruct((_NW, 10, 16), jnp.float32),
        scratch_types=scratch,
    )
    return fn(pred_flat, gt_flat, ptv_flat, oar_flat)


@jax.jit
def kernel(pred, gt, PTVs, OAR, max_dose_weight, min_dose_weight, PTV_weight):
    pred3 = pred.reshape(2, _ROWS, _LANES)
    gt4 = gt.reshape(2, 2, _ROWS, _LANES)
    ptv3 = PTVs.reshape(2, _ROWS, _LANES)
    oar4 = OAR.reshape(2, 7, _ROWS, _LANES)

    tc = _tc_partials(pred3, gt4, ptv3, oar4)
    sc = _sc_partials(pred.reshape(-1), gt.reshape(-1),
                      PTVs.reshape(-1), OAR.reshape(-1))

    s1 = tc[0] + jnp.sum(sc[:, 0, :])
    c1 = tc[1] + jnp.sum(sc[:, 1, :])
    s2 = tc[2] + jnp.sum(sc[:, 2, :])
    c2 = tc[3] + jnp.sum(sc[:, 3, :])
    s3 = tc[4] + jnp.sum(sc[:, 4, :])
    c3 = tc[5] + jnp.sum(sc[:, 5, :])
    dose_max = jnp.maximum(tc[6], jnp.max(sc[:, 6, :]))
    pred_max = jnp.maximum(tc[7], jnp.max(sc[:, 7, :]))
    dose_min = jnp.minimum(tc[8], jnp.min(sc[:, 8, :]))
    pred_min = jnp.minimum(tc[9], jnp.min(sc[:, 9, :]))

    max_pen = jnp.maximum(pred_max - dose_max, 0.0) ** 2
    min_pen = jnp.maximum(dose_min - pred_min, 0.0) ** 2
    return (s1 / c1
            + PTV_weight * (s2 / c2)
            + s3 / c3
            + max_dose_weight * max_pen
            + min_dose_weight * min_pen)


# speedup vs baseline: 2.9263x; 1.6051x over previous
import jax
import jax.numpy as jnp
from jax.experimental import pallas as pl
from jax.experimental.pallas import tpu as pltpu

_ROWS = 16384
_LANES = 128
_BR = 2048
_NB = _ROWS // _BR


def _loss_kernel(wmax_ref, wmin_ref, wptv_ref,
                 pred_ref, dose_ref, mask_ref, ptv_ref, oar_ref,
                 out_ref,
                 s_ref, mx_ref, mn_ref):
    b = pl.program_id(0)
    r = pl.program_id(1)

    @pl.when(jnp.logical_and(b == 0, r == 0))
    def _init():
        s_ref[...] = jnp.zeros_like(s_ref)
        mx_ref[...] = jnp.full_like(mx_ref, -jnp.inf)
        mn_ref[...] = jnp.full_like(mn_ref, jnp.inf)

    pred = pred_ref[0]
    dose = dose_ref[0, 0]
    mask = mask_ref[0, 0]
    ptv = ptv_ref[0]

    m = (mask > 0.0).astype(jnp.float32)
    mp = (ptv > 0.0).astype(jnp.float32)
    oar_sum = jnp.sum(oar_ref[0], axis=0)
    mo = (oar_sum > 0.0).astype(jnp.float32)

    d = jnp.abs(pred - dose)

    def tile_sum(x):
        return jnp.sum(x.reshape(_BR // 8, 8, _LANES), axis=0)

    s_ref[0] += tile_sum(d * m)
    s_ref[1] += tile_sum(m)
    s_ref[2] += tile_sum(d * mp)
    s_ref[3] += tile_sum(mp)
    s_ref[4] += tile_sum(d * mo)
    s_ref[5] += tile_sum(mo)

    neg_inf = jnp.float32(-jnp.inf)
    pos_inf = jnp.float32(jnp.inf)
    mb = mask > 0.0

    def tile_max(x):
        return jnp.max(x.reshape(_BR // 8, 8, _LANES), axis=0)

    def tile_min(x):
        return jnp.min(x.reshape(_BR // 8, 8, _LANES), axis=0)

    mx_ref[0] = jnp.maximum(mx_ref[0], tile_max(jnp.where(mb, dose, neg_inf)))
    mx_ref[1] = jnp.maximum(mx_ref[1], tile_max(jnp.where(mb, pred, neg_inf)))
    mn_ref[0] = jnp.minimum(mn_ref[0], tile_min(jnp.where(mb, dose, pos_inf)))
    mn_ref[1] = jnp.minimum(mn_ref[1], tile_min(jnp.where(mb, pred, pos_inf)))

    @pl.when(jnp.logical_and(b == pl.num_programs(0) - 1,
                             r == pl.num_programs(1) - 1))
    def _finalize():
        l1_num = jnp.sum(s_ref[0])
        l1_den = jnp.sum(s_ref[1])
        ptv_num = jnp.sum(s_ref[2])
        ptv_den = jnp.sum(s_ref[3])
        oar_num = jnp.sum(s_ref[4])
        oar_den = jnp.sum(s_ref[5])
        dose_max = jnp.max(mx_ref[0])
        pred_max = jnp.max(mx_ref[1])
        dose_min = jnp.min(mn_ref[0])
        pred_min = jnp.min(mn_ref[1])

        max_pen = jnp.maximum(pred_max - dose_max, 0.0) ** 2
        min_pen = jnp.maximum(dose_min - pred_min, 0.0) ** 2
        total = (l1_num / l1_den
                 + wptv_ref[0] * (ptv_num / ptv_den)
                 + oar_num / oar_den
                 + wmax_ref[0] * max_pen
                 + wmin_ref[0] * min_pen)
        out_ref[0] = total


@jax.jit
def kernel(pred, gt, PTVs, OAR, max_dose_weight, min_dose_weight, PTV_weight):
    pred3 = pred.reshape(2, _ROWS, _LANES)
    gt4 = gt.reshape(2, 2, _ROWS, _LANES)
    ptv3 = PTVs.reshape(2, _ROWS, _LANES)
    oar4 = OAR.reshape(2, 7, _ROWS, _LANES)

    grid = (2, _NB)

    out = pl.pallas_call(
        _loss_kernel,
        grid=grid,
        in_specs=[
            pl.BlockSpec(memory_space=pltpu.SMEM),
            pl.BlockSpec(memory_space=pltpu.SMEM),
            pl.BlockSpec(memory_space=pltpu.SMEM),
            pl.BlockSpec((1, _BR, _LANES), lambda b, r: (b, r, 0)),
            pl.BlockSpec((1, 1, _BR, _LANES), lambda b, r: (0, b, r, 0)),
            pl.BlockSpec((1, 1, _BR, _LANES), lambda b, r: (1, b, r, 0)),
            pl.BlockSpec((1, _BR, _LANES), lambda b, r: (b, r, 0)),
            pl.BlockSpec((1, 7, _BR, _LANES), lambda b, r: (b, 0, r, 0)),
        ],
        out_specs=pl.BlockSpec(memory_space=pltpu.SMEM),
        out_shape=jax.ShapeDtypeStruct((1,), jnp.float32),
        scratch_shapes=[
            pltpu.VMEM((6, 8, _LANES), jnp.float32),
            pltpu.VMEM((2, 8, _LANES), jnp.float32),
            pltpu.VMEM((2, 8, _LANES), jnp.float32),
        ],
    )(
        max_dose_weight.reshape(1), min_dose_weight.reshape(1),
        PTV_weight.reshape(1),
        pred3, gt4, gt4, ptv3, oar4,
    )
    return out[0]
